# baseline (speedup 1.0000x reference)
import functools
import math

import jax
import jax.numpy as jnp
from jax import lax
from jax.experimental import pallas as pl
from jax.experimental.pallas import tpu as pltpu
from jax.experimental.pallas import tpu_sc as plsc

N_FEATURES = 128
N_BASIS = 20
CUTOFF = 5.0
EPSILON = 1e-08


def _tc_body(d_ref, dv_ref, edge_ref, dvn_ref, dcp_ref, zf_ref, zdr_ref):
    d = d_ref[0]
    theta = d * (math.pi / CUTOFF)
    s1 = jnp.sin(theta)
    c2 = 2.0 * jnp.cos(theta)
    rinv = 1.0 / (d + EPSILON)
    s_prev = jnp.zeros_like(d)
    s_cur = s1
    edge_ref[0, 0] = s1 * rinv
    for k in range(1, N_BASIS):
        s_next = c2 * s_cur - s_prev
        s_prev, s_cur = s_cur, s_next
        edge_ref[0, k] = s_cur * rinv
    for c in range(3):
        dvn_ref[0, c] = dv_ref[0, c] * rinv
    dcp_ref[0] = d
    zf_ref[...] = jnp.zeros_like(zf_ref)
    zdr_ref[...] = jnp.zeros_like(zdr_ref)


@functools.partial(jax.jit, static_argnames=("b", "nb", "a", "f"))
def _tc_radial(d_t, dv_t, b, nb, a, f):
    sa = a // 2
    return pl.pallas_call(
        _tc_body,
        grid=(b, 2),
        in_specs=[
            pl.BlockSpec((1, nb, sa), lambda i, j: (i, 0, j)),
            pl.BlockSpec((1, 3, nb, sa), lambda i, j: (i, 0, 0, j)),
        ],
        out_specs=[
            pl.BlockSpec((1, N_BASIS, nb, sa), lambda i, j: (i, 0, 0, j)),
            pl.BlockSpec((1, 3, nb, sa), lambda i, j: (i, 0, 0, j)),
            pl.BlockSpec((1, nb, sa), lambda i, j: (i, 0, j)),
            pl.BlockSpec((1, 3, sa, f), lambda i, j: (i, 0, j, 0)),
            pl.BlockSpec((1, 3, sa, f), lambda i, j: (i, 0, j, 0)),
        ],
        out_shape=[
            jax.ShapeDtypeStruct((b, N_BASIS, nb, a), jnp.float32),
            jax.ShapeDtypeStruct((b, 3, nb, a), jnp.float32),
            jax.ShapeDtypeStruct((b, nb, a), jnp.float32),
            jax.ShapeDtypeStruct((b, 3, a, f), jnp.float32),
            jax.ShapeDtypeStruct((b, 3, a, f), jnp.float32),
        ],
    )(d_t, dv_t)


_NC, _NS = 2, 16
_NW = _NC * _NS


def _sc_gather_body(b_per_w, nz, table_hbm, idx_hbm, out_hbm, idx_v, table_v,
                    rows_v, sem):
    sid = lax.axis_index("s")
    wid = sid * _NC + lax.axis_index("c")
    base = wid * b_per_w
    @pl.when(sid == 0)
    def _stage():
        pltpu.sync_copy(table_hbm, table_v)

    pltpu.sync_copy(idx_hbm.at[pl.ds(base, b_per_w)], idx_v)
    plsc.subcore_barrier()
    pltpu.async_copy(table_v.at[idx_v], rows_v, sem).wait()
    pltpu.sync_copy(rows_v, out_hbm.at[pl.ds(base, b_per_w)])


@functools.partial(jax.jit, static_argnames=("rows", "feat"))
def _sc_gather(table, idx, rows, feat):
    nz = table.shape[0]
    b_per_w = rows // _NW
    mesh = plsc.VectorSubcoreMesh(
        core_axis_name="c", subcore_axis_name="s", num_cores=_NC, num_subcores=_NS
    )
    return pl.kernel(
        functools.partial(_sc_gather_body, b_per_w, nz),
        out_type=jax.ShapeDtypeStruct((rows, feat), jnp.float32),
        mesh=mesh,
        scratch_types=[
            pltpu.VMEM((b_per_w,), jnp.int32),
            pltpu.VMEM_SHARED((nz, feat), jnp.float32),
            pltpu.VMEM((b_per_w, feat), jnp.float32),
            pltpu.SemaphoreType.DMA,
        ],
    )(table, idx)


def kernel(atomic_numbers, positions, neighbor_mask, distances, distance_vectors,
           node_embedding_weight):
    B, A = atomic_numbers.shape
    NB = distances.shape[-1]
    F = node_embedding_weight.shape[-1]
    rows = B * A

    idx = atomic_numbers.reshape(rows).astype(jnp.int32)
    inv_node = _sc_gather(node_embedding_weight, idx, rows=rows, feat=F)

    d_t = jnp.transpose(distances, (0, 2, 1))
    dv_t = jnp.transpose(distance_vectors, (0, 3, 2, 1))
    edge_t, dvn_t, dcp_t, zf_t, zdr_t = _tc_radial(d_t, dv_t, b=B, nb=NB, a=A, f=F)

    invariant_node = inv_node.reshape(B, A, F)
    invariant_edge = jnp.transpose(edge_t, (0, 3, 2, 1))
    dvn = jnp.transpose(dvn_t, (0, 3, 2, 1))
    d_out = jnp.transpose(dcp_t, (0, 2, 1))
    eq_F = jnp.zeros((B, A, 3), jnp.float32)
    eq_f = jnp.transpose(zf_t, (0, 2, 1, 3))
    eq_dr = jnp.transpose(zdr_t, (0, 2, 1, 3))
    return (invariant_node, eq_F, eq_f, eq_dr, invariant_edge, d_out, dvn)


# speedup vs baseline: 1.0992x; 1.0992x over previous
import functools
import math

import jax
import jax.numpy as jnp
from jax import lax
from jax.experimental import pallas as pl
from jax.experimental.pallas import tpu as pltpu
from jax.experimental.pallas import tpu_sc as plsc

N_FEATURES = 128
N_BASIS = 20
CUTOFF = 5.0
EPSILON = 1e-08


def _tc_body(d_ref, dv_ref, edge_ref, dvn_ref, dcp_ref, zf_ref, zdr_ref):
    d = d_ref[0]
    theta = d * (math.pi / CUTOFF)
    s1 = jnp.sin(theta)
    c2 = 2.0 * jnp.cos(theta)
    rinv = 1.0 / (d + EPSILON)
    s_prev = jnp.zeros_like(d)
    s_cur = s1
    edge_ref[0, 0] = s1 * rinv
    for k in range(1, N_BASIS):
        s_next = c2 * s_cur - s_prev
        s_prev, s_cur = s_cur, s_next
        edge_ref[0, k] = s_cur * rinv
    for c in range(3):
        dvn_ref[0, c] = dv_ref[0, c] * rinv
    dcp_ref[0] = d
    zf_ref[...] = jnp.zeros_like(zf_ref)
    zdr_ref[...] = jnp.zeros_like(zdr_ref)


@functools.partial(jax.jit, static_argnames=("b", "nb", "a", "f"))
def _tc_radial(d_t, dv_t, b, nb, a, f):
    return pl.pallas_call(
        _tc_body,
        grid=(b,),
        in_specs=[
            pl.BlockSpec((1, nb, a), lambda i: (i, 0, 0)),
            pl.BlockSpec((1, 3, nb, a), lambda i: (i, 0, 0, 0)),
        ],
        out_specs=[
            pl.BlockSpec((1, N_BASIS, nb, a), lambda i: (i, 0, 0, 0)),
            pl.BlockSpec((1, 3, nb, a), lambda i: (i, 0, 0, 0)),
            pl.BlockSpec((1, nb, a), lambda i: (i, 0, 0)),
            pl.BlockSpec((1, 3, a, f), lambda i: (i, 0, 0, 0)),
            pl.BlockSpec((1, 3, a, f), lambda i: (i, 0, 0, 0)),
        ],
        out_shape=[
            jax.ShapeDtypeStruct((b, N_BASIS, nb, a), jnp.float32),
            jax.ShapeDtypeStruct((b, 3, nb, a), jnp.float32),
            jax.ShapeDtypeStruct((b, nb, a), jnp.float32),
            jax.ShapeDtypeStruct((b, 3, a, f), jnp.float32),
            jax.ShapeDtypeStruct((b, 3, a, f), jnp.float32),
        ],
    )(d_t, dv_t)


_NC, _NS = 1, 16
_NW = _NC * _NS


def _sc_gather_body(b_per_w, nz, table_hbm, idx_hbm, out_hbm, idx_a, idx_b,
                    table_v, rows_v, sem):
    sid = lax.axis_index("s")
    wid = sid * _NC + lax.axis_index("c")
    base = wid * b_per_w
    half = b_per_w // 2
    @pl.when(sid == 0)
    def _stage():
        pltpu.sync_copy(table_hbm, table_v)

    pltpu.sync_copy(idx_hbm.at[pl.ds(base, half)], idx_a)
    pltpu.sync_copy(idx_hbm.at[pl.ds(base + half, half)], idx_b)
    plsc.subcore_barrier()
    pltpu.async_copy(table_v.at[idx_a], rows_v, sem).wait()
    pltpu.sync_copy(rows_v, out_hbm.at[pl.ds(base, half)])
    pltpu.async_copy(table_v.at[idx_b], rows_v, sem).wait()
    pltpu.sync_copy(rows_v, out_hbm.at[pl.ds(base + half, half)])


@functools.partial(jax.jit, static_argnames=("rows", "feat"))
def _sc_gather(table, idx, rows, feat):
    nz = table.shape[0]
    b_per_w = rows // _NW
    mesh = plsc.VectorSubcoreMesh(
        core_axis_name="c", subcore_axis_name="s", num_cores=_NC, num_subcores=_NS
    )
    return pl.kernel(
        functools.partial(_sc_gather_body, b_per_w, nz),
        out_type=jax.ShapeDtypeStruct((rows, feat), jnp.float32),
        mesh=mesh,
        scratch_types=[
            pltpu.VMEM((b_per_w // 2,), jnp.int32),
            pltpu.VMEM((b_per_w // 2,), jnp.int32),
            pltpu.VMEM_SHARED((nz, feat), jnp.float32),
            pltpu.VMEM((b_per_w // 2, feat), jnp.float32),
            pltpu.SemaphoreType.DMA,
        ],
    )(table, idx)


def kernel(atomic_numbers, positions, neighbor_mask, distances, distance_vectors,
           node_embedding_weight):
    B, A = atomic_numbers.shape
    NB = distances.shape[-1]
    F = node_embedding_weight.shape[-1]
    rows = B * A

    idx = atomic_numbers.reshape(rows).astype(jnp.int32)
    inv_node = _sc_gather(node_embedding_weight, idx, rows=rows, feat=F)

    d_t = jnp.transpose(distances, (0, 2, 1))
    dv_t = jnp.transpose(distance_vectors, (0, 3, 2, 1))
    edge_t, dvn_t, dcp_t, zf_t, zdr_t = _tc_radial(d_t, dv_t, b=B, nb=NB, a=A, f=F)

    invariant_node = inv_node.reshape(B, A, F)
    invariant_edge = jnp.transpose(edge_t, (0, 3, 2, 1))
    dvn = jnp.transpose(dvn_t, (0, 3, 2, 1))
    d_out = jnp.transpose(dcp_t, (0, 2, 1))
    eq_F = jnp.zeros((B, A, 3), jnp.float32)
    eq_f = jnp.transpose(zf_t, (0, 2, 1, 3))
    eq_dr = jnp.transpose(zdr_t, (0, 2, 1, 3))
    return (invariant_node, eq_F, eq_f, eq_dr, invariant_edge, d_out, dvn)
